# speedup vs baseline: 3.3269x; 3.3269x over previous
import functools

import jax
import jax.numpy as jnp
from jax import lax
from jax.experimental import pallas as pl
from jax.experimental.pallas import tpu as pltpu
from jax.experimental.pallas import tpu_sc as plsc

_NC = 2
_NS = 16
_NW = _NC * _NS
_EB = 128
_GRAPHS = 128
_RB = 1000


def _make_sc_agg(npad, nb, emb):
    chunk = npad // _NS
    mesh = plsc.VectorSubcoreMesh(
        core_axis_name="c", subcore_axis_name="s",
        num_cores=_NC, num_subcores=_NS)

    @functools.partial(
        pl.kernel,
        out_type=jax.ShapeDtypeStruct((_NC, npad, emb), jnp.float32),
        mesh=mesh,
        scratch_types=[
            pltpu.VMEM((nb, _EB), jnp.int32),
            pltpu.VMEM((nb, _EB), jnp.int32),
            pltpu.VMEM((_EB, emb), jnp.float32),
            pltpu.VMEM_SHARED((npad, emb), jnp.float32),
            pltpu.SemaphoreType.DMA,
        ],
    )
    def k(table_hbm, src_hbm, dst_hbm, zeros_hbm, out_hbm,
          src_v, dst_v, rows_v, acc_sh, sem):
        c = lax.axis_index("c")
        s = lax.axis_index("s")
        wid = s * _NC + c
        pltpu.sync_copy(zeros_hbm.at[pl.ds(s * chunk, chunk)],
                        acc_sh.at[pl.ds(s * chunk, chunk)])
        pltpu.sync_copy(src_hbm.at[wid], src_v)
        pltpu.sync_copy(dst_hbm.at[wid], dst_v)
        plsc.subcore_barrier()

        def body(j, carry):
            pltpu.async_copy(table_hbm.at[src_v.at[j]], rows_v, sem).wait()
            pltpu.sync_copy(rows_v, acc_sh.at[dst_v.at[j]], add=True)
            return carry

        lax.fori_loop(0, nb, body, 0)
        plsc.subcore_barrier()
        pltpu.sync_copy(acc_sh.at[pl.ds(s * chunk, chunk)],
                        out_hbm.at[c].at[pl.ds(s * chunk, chunk)])

    return k


def _embed_body(x_ref, ea_ref, ec_ref, o_ref):
    io = lax.broadcasted_iota(jnp.int32, (_RB, 128), 1)
    oha = (x_ref[:, 0:1] == io).astype(jnp.float32)
    ohc = (x_ref[:, 1:2] == io).astype(jnp.float32)
    o_ref[...] = (
        jnp.dot(oha, ea_ref[...], preferred_element_type=jnp.float32)
        + jnp.dot(ohc, ec_ref[...], preferred_element_type=jnp.float32))


def _layer_body(relu, h_ref, a_ref, cw_ref, w_ref, m_ref, v_ref, o_ref):
    xs = a_ref[0] + a_ref[1] + h_ref[...]
    z = jnp.dot(xs, w_ref[...], preferred_element_type=jnp.float32)
    z = z + jnp.dot(cw_ref[0] + cw_ref[1], m_ref[...],
                    preferred_element_type=jnp.float32)
    z = (z + v_ref[0:1]) * v_ref[1:2] + v_ref[2:3]
    if relu:
        z = jnp.maximum(z, 0.0)
    o_ref[...] = z


def _pool_body(nsteps, b_ref, h_ref, wf_ref, bf_ref, o_ref, acc_s, cnt_s):
    i = pl.program_id(0)

    @pl.when(i == 0)
    def _():
        acc_s[...] = jnp.zeros_like(acc_s)
        cnt_s[...] = jnp.zeros_like(cnt_s)

    io = lax.broadcasted_iota(jnp.int32, (_RB, _GRAPHS), 1)
    oh = (b_ref[...] == io).astype(jnp.float32)
    dn = (((0,), (0,)), ((), ()))
    acc_s[...] += lax.dot_general(oh, h_ref[...], dn,
                                  preferred_element_type=jnp.float32)
    cnt_s[...] += lax.dot_general(oh, jnp.ones((_RB, 128), jnp.float32), dn,
                                  preferred_element_type=jnp.float32)

    @pl.when(i == nsteps - 1)
    def _():
        pooled = acc_s[...] / jnp.maximum(cnt_s[...], 1.0)
        o_ref[...] = (jnp.dot(pooled, wf_ref[...],
                              preferred_element_type=jnp.float32)
                      + bf_ref[...])


def kernel(x, edge_index, edge_attr, batch, emb_atom, emb_chir, Ws, bs,
           ee1, ee2, bn_gamma, bn_beta, W_feat, b_feat):
    n = x.shape[0]
    e = edge_index.shape[1]
    emb = emb_atom.shape[1]
    feat = W_feat.shape[1]
    nlayer = Ws.shape[0]
    npad = ((n + _NS * 8 - 1) // (_NS * 8) + 1) * (_NS * 8)
    nb = -(-e // (_NW * _EB))
    etot = _NW * nb * _EB
    grid_n = n // _RB

    pad = etot - e
    src = jnp.concatenate([edge_index[0], jnp.zeros((pad,), jnp.int32)])
    dst = jnp.concatenate([edge_index[1],
                           jnp.full((pad,), npad - 1, jnp.int32)])
    kc = edge_attr[:, 0] * 3 + edge_attr[:, 1]
    srcc = jnp.concatenate([kc, jnp.zeros((pad,), jnp.int32)])
    src3 = src.reshape(_NW, nb, _EB)
    dst3 = dst.reshape(_NW, nb, _EB)
    srcc3 = srcc.reshape(_NW, nb, _EB)
    zeros = jnp.zeros((npad, emb), jnp.float32)
    eye16 = jnp.eye(16, emb, dtype=jnp.float32)

    ea_pad = jnp.zeros((128, emb), jnp.float32).at[:emb_atom.shape[0]].set(emb_atom)
    ec_pad = jnp.zeros((128, emb), jnp.float32).at[:emb_chir.shape[0]].set(emb_chir)
    ki = jnp.arange(15)
    lut = ee1[:, ki // 3, 0] + ee2[:, ki % 3, 0]
    M = jnp.zeros((nlayer, 128, emb), jnp.float32).at[:, :15, :].set(
        lut[:, :, None] * jnp.ones((1, 1, emb), jnp.float32))
    beff = bs + (ee1[:, 4, 0] + ee2[:, 0, 0])[:, None]
    vconsts = jnp.stack([beff, bn_gamma, bn_beta], axis=1)

    rows = lambda shape: pl.BlockSpec(
        shape, lambda i: (i,) + (0,) * (len(shape) - 1))
    full = lambda shape: pl.BlockSpec(shape, lambda i: (0,) * len(shape))
    mid = pl.BlockSpec((_NC, _RB, emb), lambda i: (0, i, 0))

    h = pl.pallas_call(
        _embed_body,
        grid=(grid_n,),
        in_specs=[rows((_RB, 2)), full((128, emb)), full((128, emb))],
        out_specs=rows((_RB, emb)),
        out_shape=jax.ShapeDtypeStruct((n, emb), jnp.float32),
    )(x, ea_pad, ec_pad)

    sc_agg = _make_sc_agg(npad, nb, emb)
    cw = sc_agg(eye16, srcc3, dst3, zeros)

    def layer_call(relu):
        return pl.pallas_call(
            functools.partial(_layer_body, relu),
            grid=(grid_n,),
            in_specs=[rows((_RB, emb)), mid, mid, full((emb, emb)),
                      full((128, emb)), full((3, emb))],
            out_specs=rows((_RB, emb)),
            out_shape=jax.ShapeDtypeStruct((n, emb), jnp.float32),
        )

    for l in range(nlayer):
        a = sc_agg(h, src3, dst3, zeros)
        h = layer_call(l < nlayer - 1)(h, a, cw, Ws[l], M[l], vconsts[l])

    out = pl.pallas_call(
        functools.partial(_pool_body, grid_n),
        grid=(grid_n,),
        in_specs=[rows((_RB, 1)), rows((_RB, emb)),
                  full((emb, feat)), full((1, feat))],
        out_specs=full((_GRAPHS, feat)),
        out_shape=jax.ShapeDtypeStruct((_GRAPHS, feat), jnp.float32),
        scratch_shapes=[pltpu.VMEM((_GRAPHS, emb), jnp.float32),
                        pltpu.VMEM((_GRAPHS, 128), jnp.float32)],
    )(batch.reshape(n, 1), h, W_feat, b_feat.reshape(1, feat))
    return out
